# baseline (speedup 1.0000x reference)
import functools

import jax
import jax.numpy as jnp
from jax import lax
from jax.experimental import pallas as pl
from jax.experimental.pallas import tpu as pltpu
from jax.experimental.pallas import tpu_sc as plsc

DIM = 33
NLUT = DIM * DIM * DIM
NPAD = 35944
N_IMG = 16
HW = 512 * 512
PIX = N_IMG * HW
NW = 32
PPW = PIX // NW
CHUNK = 1024
NCHUNK = PPW // CHUNK
L = 16

_BINSIZE = 1.000001 / (DIM - 1)
_S = float(1.0 / _BINSIZE)


def _tec_body(lut_hbm, x_hbm, out_hbm, lut0, lut1, lut2,
              r0, g0, b0, r1, g1, b1,
              o00, o10, o20, o01, o11, o21,
              isem0, isem1, osem0, osem1):
    cid = lax.axis_index("c")
    sid = lax.axis_index("s")
    wid = sid * 2 + cid
    img = wid // 2
    half = wid % 2
    base = half * PPW

    pltpu.sync_copy(lut_hbm.at[pl.ds(0, NPAD)], lut0)
    pltpu.sync_copy(lut_hbm.at[pl.ds(NPAD, NPAD)], lut1)
    pltpu.sync_copy(lut_hbm.at[pl.ds(2 * NPAD, NPAD)], lut2)

    luts = (lut0, lut1, lut2)
    inbufs = ((r0, g0, b0), (r1, g1, b1))
    outbufs = ((o00, o10, o20), (o01, o11, o21))
    isems = (isem0, isem1)
    osems = (osem0, osem1)

    def issue_in(g, b):
        off = base + g * CHUNK
        for ch, buf in enumerate(inbufs[b]):
            pltpu.async_copy(
                x_hbm.at[pl.ds((img * 3 + ch) * HW + off, CHUNK)], buf,
                isems[b])

    def wait_in(b):
        for buf in inbufs[b]:
            pltpu.make_async_copy(
                x_hbm.at[pl.ds(0, CHUNK)], buf, isems[b]).wait()

    def issue_out(g, b):
        off = base + g * CHUNK
        for ch, buf in enumerate(outbufs[b]):
            pltpu.async_copy(
                buf, out_hbm.at[pl.ds((img * 3 + ch) * HW + off, CHUNK)],
                osems[b])

    def wait_out(b):
        for buf in outbufs[b]:
            pltpu.make_async_copy(
                buf, out_hbm.at[pl.ds(0, CHUNK)], osems[b]).wait()

    def compute(b):
        rbuf, gbuf, bbuf = inbufs[b]
        obufs = outbufs[b]

        @plsc.parallel_loop(0, CHUNK, step=L, unroll=4)
        def vbody(p):
            rv = rbuf[pl.ds(p, L)]
            gv = gbuf[pl.ds(p, L)]
            bv = bbuf[pl.ds(p, L)]
            tr = rv * _S
            tg = gv * _S
            tb = bv * _S
            ir = tr.astype(jnp.int32)
            ig = tg.astype(jnp.int32)
            ib = tb.astype(jnp.int32)
            rd = tr - ir.astype(jnp.float32)
            gd = tg - ig.astype(jnp.float32)
            bd = tb - ib.astype(jnp.float32)
            cell = ir + ig * DIM + ib * (DIM * DIM)
            wg0 = 1.0 - gd
            wb0 = 1.0 - bd
            w00 = wg0 * wb0
            w10 = gd * wb0
            w01 = wg0 * bd
            w11 = gd * bd
            acc = [None, None, None]
            for off, w in ((0, w00), (DIM, w10),
                           (DIM * DIM, w01), (DIM * DIM + DIM, w11)):
                i0 = cell + off if off else cell
                i1 = i0 + 1
                for c, lut in enumerate(luts):
                    a = plsc.load_gather(lut, [i0])
                    bb = plsc.load_gather(lut, [i1])
                    v = a + rd * (bb - a)
                    acc[c] = w * v if acc[c] is None else acc[c] + w * v
            for c, ob in enumerate(obufs):
                ob[pl.ds(p, L)] = acc[c]

    issue_in(0, 0)

    def pair_body(G, _):
        for b in (0, 1):
            g = 2 * G + b

            @pl.when(g + 1 < NCHUNK)
            def _():
                issue_in(g + 1, 1 - b)

            wait_in(b)

            @pl.when(g >= 2)
            def _():
                wait_out(b)

            compute(b)
            issue_out(g, b)
        return ()

    lax.fori_loop(0, NCHUNK // 2, pair_body, ())
    wait_out(0)
    wait_out(1)


_lut3d = functools.partial(
    pl.kernel,
    out_type=jax.ShapeDtypeStruct((N_IMG * 3 * HW,), jnp.float32),
    mesh=plsc.VectorSubcoreMesh(core_axis_name="c", subcore_axis_name="s"),
    scratch_types=(
        [pltpu.VMEM((NPAD,), jnp.float32)] * 3
        + [pltpu.VMEM((CHUNK,), jnp.float32)] * 12
        + [pltpu.SemaphoreType.DMA] * 4
    ),
    compiler_params=pltpu.CompilerParams(needs_layout_passes=False),
)(_tec_body)


@jax.jit
def kernel(LUT, x):
    lut_flat = LUT.reshape(3, NLUT)
    lut_pad = jnp.pad(lut_flat, ((0, 0), (0, NPAD - NLUT))).reshape(-1)
    xf = x.reshape(-1)
    out = _lut3d(lut_pad, xf)
    return out.reshape(N_IMG, 3, 512, 512)


# speedup vs baseline: 1.0157x; 1.0157x over previous
import functools

import jax
import jax.numpy as jnp
from jax import lax
from jax.experimental import pallas as pl
from jax.experimental.pallas import tpu as pltpu
from jax.experimental.pallas import tpu_sc as plsc

DIM = 33
NLUT = DIM * DIM * DIM
NPAD = 35944
N_IMG = 16
HW = 512 * 512
PIX = N_IMG * HW
NW = 32
PPW = PIX // NW
CHUNK = 2048
NCHUNK = PPW // CHUNK
L = 16

_BINSIZE = 1.000001 / (DIM - 1)
_S = float(1.0 / _BINSIZE)


def _tec_body(lut_hbm, x_hbm, out_hbm, lut0, lut1, lut2,
              r0, g0, b0, r1, g1, b1, r2, g2, b2,
              isem0, isem1, isem2, osem0, osem1, osem2):
    cid = lax.axis_index("c")
    sid = lax.axis_index("s")
    wid = sid * 2 + cid
    img = wid // 2
    half = wid % 2
    base = half * PPW

    pltpu.sync_copy(lut_hbm.at[pl.ds(0, NPAD)], lut0)
    pltpu.sync_copy(lut_hbm.at[pl.ds(NPAD, NPAD)], lut1)
    pltpu.sync_copy(lut_hbm.at[pl.ds(2 * NPAD, NPAD)], lut2)

    luts = (lut0, lut1, lut2)
    bufsets = ((r0, g0, b0), (r1, g1, b1), (r2, g2, b2))
    isems = (isem0, isem1, isem2)
    osems = (osem0, osem1, osem2)

    def issue_in(g, s):
        off = base + g * CHUNK
        for ch, buf in enumerate(bufsets[s]):
            pltpu.async_copy(
                x_hbm.at[pl.ds((img * 3 + ch) * HW + off, CHUNK)], buf,
                isems[s])

    def wait_in(s):
        for buf in bufsets[s]:
            pltpu.make_async_copy(
                x_hbm.at[pl.ds(0, CHUNK)], buf, isems[s]).wait()

    def issue_out(g, s):
        off = base + g * CHUNK
        for ch, buf in enumerate(bufsets[s]):
            pltpu.async_copy(
                buf, out_hbm.at[pl.ds((img * 3 + ch) * HW + off, CHUNK)],
                osems[s])

    def wait_out(s):
        for buf in bufsets[s]:
            pltpu.make_async_copy(
                buf, out_hbm.at[pl.ds(0, CHUNK)], osems[s]).wait()

    def compute(s):
        rbuf, gbuf, bbuf = bufsets[s]

        @plsc.parallel_loop(0, CHUNK, step=L, unroll=3)
        def vbody(p):
            rv = rbuf[pl.ds(p, L)]
            gv = gbuf[pl.ds(p, L)]
            bv = bbuf[pl.ds(p, L)]
            tr = rv * _S
            tg = gv * _S
            tb = bv * _S
            ir = tr.astype(jnp.int32)
            ig = tg.astype(jnp.int32)
            ib = tb.astype(jnp.int32)
            rd = tr - ir.astype(jnp.float32)
            gd = tg - ig.astype(jnp.float32)
            bd = tb - ib.astype(jnp.float32)
            cell = ir + ig * DIM + ib * (DIM * DIM)
            wg0 = 1.0 - gd
            wb0 = 1.0 - bd
            w00 = wg0 * wb0
            w10 = gd * wb0
            w01 = wg0 * bd
            w11 = gd * bd
            acc = [None, None, None]
            for off, w in ((0, w00), (DIM, w10),
                           (DIM * DIM, w01), (DIM * DIM + DIM, w11)):
                i0 = cell + off if off else cell
                i1 = i0 + 1
                for c, lut in enumerate(luts):
                    a = plsc.load_gather(lut, [i0])
                    bb = plsc.load_gather(lut, [i1])
                    v = a + rd * (bb - a)
                    acc[c] = w * v if acc[c] is None else acc[c] + w * v
            rbuf[pl.ds(p, L)] = acc[0]
            gbuf[pl.ds(p, L)] = acc[1]
            bbuf[pl.ds(p, L)] = acc[2]

    def step(g, s):
        wait_in(s)
        compute(s)
        issue_out(g, s)
        s2 = (s + 2) % 3

        @pl.when(g >= 1)
        def _():
            wait_out(s2)

        @pl.when(g + 2 < NCHUNK)
        def _():
            issue_in(g + 2, s2)

    issue_in(0, 0)
    issue_in(1, 1)

    def triple_body(T, _):
        for s in (0, 1, 2):
            step(3 * T + s, s)
        return ()

    lax.fori_loop(0, (NCHUNK - 1) // 3, triple_body, ())
    step(NCHUNK - 1, (NCHUNK - 1) % 3)
    wait_out((NCHUNK - 1) % 3)


_lut3d = functools.partial(
    pl.kernel,
    out_type=jax.ShapeDtypeStruct((N_IMG * 3 * HW,), jnp.float32),
    mesh=plsc.VectorSubcoreMesh(core_axis_name="c", subcore_axis_name="s"),
    scratch_types=(
        [pltpu.VMEM((NPAD,), jnp.float32)] * 3
        + [pltpu.VMEM((CHUNK,), jnp.float32)] * 9
        + [pltpu.SemaphoreType.DMA] * 6
    ),
    compiler_params=pltpu.CompilerParams(needs_layout_passes=False),
)(_tec_body)


@jax.jit
def kernel(LUT, x):
    lut_flat = LUT.reshape(3, NLUT)
    lut_pad = jnp.pad(lut_flat, ((0, 0), (0, NPAD - NLUT))).reshape(-1)
    xf = x.reshape(-1)
    out = _lut3d(lut_pad, xf)
    return out.reshape(N_IMG, 3, 512, 512)


# speedup vs baseline: 1.0190x; 1.0033x over previous
import functools

import jax
import jax.numpy as jnp
from jax import lax
from jax.experimental import pallas as pl
from jax.experimental.pallas import tpu as pltpu
from jax.experimental.pallas import tpu_sc as plsc

DIM = 33
NLUT = DIM * DIM * DIM
NPAD = 35944
N_IMG = 16
HW = 512 * 512
PIX = N_IMG * HW
NW = 32
PPW = PIX // NW
CHUNK = 1024
NCHUNK = PPW // CHUNK
L = 16

_BINSIZE = 1.000001 / (DIM - 1)
_S = float(1.0 / _BINSIZE)


def _tec_body(lut_hbm, x_hbm, out_hbm, lut0, lut1, lut2,
              r0, g0, b0, r1, g1, b1,
              o00, o10, o20, o01, o11, o21,
              isem0, isem1, osem0, osem1):
    cid = lax.axis_index("c")
    sid = lax.axis_index("s")
    wid = sid * 2 + cid
    img = wid // 2
    half = wid % 2
    base = half * PPW

    pltpu.sync_copy(lut_hbm.at[pl.ds(0, NPAD)], lut0)
    pltpu.sync_copy(lut_hbm.at[pl.ds(NPAD, NPAD)], lut1)
    pltpu.sync_copy(lut_hbm.at[pl.ds(2 * NPAD, NPAD)], lut2)

    luts = (lut0, lut1, lut2)
    inbufs = ((r0, g0, b0), (r1, g1, b1))
    outbufs = ((o00, o10, o20), (o01, o11, o21))
    isems = (isem0, isem1)
    osems = (osem0, osem1)

    def issue_in(g, b):
        off = base + g * CHUNK
        for ch, buf in enumerate(inbufs[b]):
            pltpu.async_copy(
                x_hbm.at[pl.ds((img * 3 + ch) * HW + off, CHUNK)], buf,
                isems[b])

    def wait_in(b):
        for buf in inbufs[b]:
            pltpu.make_async_copy(
                x_hbm.at[pl.ds(0, CHUNK)], buf, isems[b]).wait()

    def issue_out(g, b):
        off = base + g * CHUNK
        for ch, buf in enumerate(outbufs[b]):
            pltpu.async_copy(
                buf, out_hbm.at[pl.ds((img * 3 + ch) * HW + off, CHUNK)],
                osems[b])

    def wait_out(b):
        for buf in outbufs[b]:
            pltpu.make_async_copy(
                buf, out_hbm.at[pl.ds(0, CHUNK)], osems[b]).wait()

    def compute(b):
        rbuf, gbuf, bbuf = inbufs[b]
        obufs = outbufs[b]

        @plsc.parallel_loop(0, CHUNK, step=L, unroll=3)
        def vbody(p):
            rv = rbuf[pl.ds(p, L)]
            gv = gbuf[pl.ds(p, L)]
            bv = bbuf[pl.ds(p, L)]
            tr = rv * _S
            tg = gv * _S
            tb = bv * _S
            ir = tr.astype(jnp.int32)
            ig = tg.astype(jnp.int32)
            ib = tb.astype(jnp.int32)
            rd = tr - ir.astype(jnp.float32)
            gd = tg - ig.astype(jnp.float32)
            bd = tb - ib.astype(jnp.float32)
            cell = ir + ig * DIM + ib * (DIM * DIM)
            wg0 = 1.0 - gd
            wb0 = 1.0 - bd
            w00 = wg0 * wb0
            w10 = gd * wb0
            w01 = wg0 * bd
            w11 = gd * bd
            acc = [None, None, None]
            for off, w in ((0, w00), (DIM, w10),
                           (DIM * DIM, w01), (DIM * DIM + DIM, w11)):
                i0 = cell + off if off else cell
                i1 = i0 + 1
                for c, lut in enumerate(luts):
                    a = plsc.load_gather(lut, [i0])
                    bb = plsc.load_gather(lut, [i1])
                    v = a + rd * (bb - a)
                    acc[c] = w * v if acc[c] is None else acc[c] + w * v
            for c, ob in enumerate(obufs):
                ob[pl.ds(p, L)] = acc[c]

    issue_in(0, 0)

    def pair_body(G, _):
        for b in (0, 1):
            g = 2 * G + b

            @pl.when(g + 1 < NCHUNK)
            def _():
                issue_in(g + 1, 1 - b)

            wait_in(b)

            @pl.when(g >= 2)
            def _():
                wait_out(b)

            compute(b)
            issue_out(g, b)
        return ()

    lax.fori_loop(0, NCHUNK // 2, pair_body, ())
    wait_out(0)
    wait_out(1)


_lut3d = functools.partial(
    pl.kernel,
    out_type=jax.ShapeDtypeStruct((N_IMG * 3 * HW,), jnp.float32),
    mesh=plsc.VectorSubcoreMesh(core_axis_name="c", subcore_axis_name="s"),
    scratch_types=(
        [pltpu.VMEM((NPAD,), jnp.float32)] * 3
        + [pltpu.VMEM((CHUNK,), jnp.float32)] * 12
        + [pltpu.SemaphoreType.DMA] * 4
    ),
    compiler_params=pltpu.CompilerParams(needs_layout_passes=False),
)(_tec_body)


@jax.jit
def kernel(LUT, x):
    lut_flat = LUT.reshape(3, NLUT)
    lut_pad = jnp.pad(lut_flat, ((0, 0), (0, NPAD - NLUT))).reshape(-1)
    xf = x.reshape(-1)
    out = _lut3d(lut_pad, xf)
    return out.reshape(N_IMG, 3, 512, 512)


# speedup vs baseline: 1.0430x; 1.0236x over previous
import functools

import jax
import jax.numpy as jnp
from jax import lax
from jax.experimental import pallas as pl
from jax.experimental.pallas import tpu as pltpu
from jax.experimental.pallas import tpu_sc as plsc

DIM = 33
NLUT = DIM * DIM * DIM
NPAD = 35944
N_IMG = 16
HW = 512 * 512
PIX = N_IMG * HW
NW = 32
PPW = PIX // NW
CHUNK = 1024
NCHUNK = PPW // CHUNK
L = 16

_BINSIZE = 1.000001 / (DIM - 1)
_S = float(1.0 / _BINSIZE)


def _tec_body(lut_hbm, x_hbm, out_hbm, lut0, lut1, lut2,
              r0, g0, b0, r1, g1, b1,
              o00, o10, o20, o01, o11, o21,
              isem0, isem1, osem0, osem1):
    cid = lax.axis_index("c")
    sid = lax.axis_index("s")
    wid = sid * 2 + cid
    img = wid // 2
    half = wid % 2
    base = half * PPW

    pltpu.sync_copy(lut_hbm.at[pl.ds(0, NPAD)], lut0)
    pltpu.sync_copy(lut_hbm.at[pl.ds(NPAD, NPAD)], lut1)
    pltpu.sync_copy(lut_hbm.at[pl.ds(2 * NPAD, NPAD)], lut2)

    luts = (lut0, lut1, lut2)
    inbufs = ((r0, g0, b0), (r1, g1, b1))
    outbufs = ((o00, o10, o20), (o01, o11, o21))
    isems = (isem0, isem1)
    osems = (osem0, osem1)

    def issue_in(g, b):
        off = base + g * CHUNK
        for ch, buf in enumerate(inbufs[b]):
            pltpu.async_copy(
                x_hbm.at[pl.ds((img * 3 + ch) * HW + off, CHUNK)], buf,
                isems[b])

    def wait_in(b):
        for buf in inbufs[b]:
            pltpu.make_async_copy(
                x_hbm.at[pl.ds(0, CHUNK)], buf, isems[b]).wait()

    def issue_out(g, b):
        off = base + g * CHUNK
        for ch, buf in enumerate(outbufs[b]):
            pltpu.async_copy(
                buf, out_hbm.at[pl.ds((img * 3 + ch) * HW + off, CHUNK)],
                osems[b])

    def wait_out(b):
        for buf in outbufs[b]:
            pltpu.make_async_copy(
                buf, out_hbm.at[pl.ds(0, CHUNK)], osems[b]).wait()

    def compute(b):
        rbuf, gbuf, bbuf = inbufs[b]
        obufs = outbufs[b]

        @plsc.parallel_loop(0, CHUNK, step=L, unroll=2)
        def vbody(p):
            rv = rbuf[pl.ds(p, L)]
            gv = gbuf[pl.ds(p, L)]
            bv = bbuf[pl.ds(p, L)]
            tr = rv * _S
            tg = gv * _S
            tb = bv * _S
            ir = tr.astype(jnp.int32)
            ig = tg.astype(jnp.int32)
            ib = tb.astype(jnp.int32)
            rd = tr - ir.astype(jnp.float32)
            gd = tg - ig.astype(jnp.float32)
            bd = tb - ib.astype(jnp.float32)
            cell = ir + ig * DIM + ib * (DIM * DIM)
            wg0 = 1.0 - gd
            wb0 = 1.0 - bd
            w00 = wg0 * wb0
            w10 = gd * wb0
            w01 = wg0 * bd
            w11 = gd * bd
            acc = [None, None, None]
            for off, w in ((0, w00), (DIM, w10),
                           (DIM * DIM, w01), (DIM * DIM + DIM, w11)):
                i0 = cell + off if off else cell
                i1 = i0 + 1
                for c, lut in enumerate(luts):
                    a = plsc.load_gather(lut, [i0])
                    bb = plsc.load_gather(lut, [i1])
                    v = a + rd * (bb - a)
                    acc[c] = w * v if acc[c] is None else acc[c] + w * v
            for c, ob in enumerate(obufs):
                ob[pl.ds(p, L)] = acc[c]

    issue_in(0, 0)

    def pair_body(G, _):
        for b in (0, 1):
            g = 2 * G + b

            @pl.when(g + 1 < NCHUNK)
            def _():
                issue_in(g + 1, 1 - b)

            wait_in(b)

            @pl.when(g >= 2)
            def _():
                wait_out(b)

            compute(b)
            issue_out(g, b)
        return ()

    lax.fori_loop(0, NCHUNK // 2, pair_body, ())
    wait_out(0)
    wait_out(1)


_lut3d = functools.partial(
    pl.kernel,
    out_type=jax.ShapeDtypeStruct((N_IMG * 3 * HW,), jnp.float32),
    mesh=plsc.VectorSubcoreMesh(core_axis_name="c", subcore_axis_name="s"),
    scratch_types=(
        [pltpu.VMEM((NPAD,), jnp.float32)] * 3
        + [pltpu.VMEM((CHUNK,), jnp.float32)] * 12
        + [pltpu.SemaphoreType.DMA] * 4
    ),
    compiler_params=pltpu.CompilerParams(needs_layout_passes=False),
)(_tec_body)


@jax.jit
def kernel(LUT, x):
    lut_flat = LUT.reshape(3, NLUT)
    lut_pad = jnp.pad(lut_flat, ((0, 0), (0, NPAD - NLUT))).reshape(-1)
    xf = x.reshape(-1)
    out = _lut3d(lut_pad, xf)
    return out.reshape(N_IMG, 3, 512, 512)
